# baseline (speedup 1.0000x reference)
import functools

import jax
import jax.numpy as jnp
from jax import lax
from jax.experimental import pallas as pl
from jax.experimental.pallas import tpu as pltpu
from jax.experimental.pallas import tpu_sc as plsc

B = 4096
P = 36864
PT = 2048
NP = P // PT

SC_CORES = 2
SC_SUBCORES = 16
NW = SC_CORES * SC_SUBCORES
B_PER_W = B // NW


def _tile_body(a_ref, t_ref, iota_ref, m_ref, l_ref):
    s = jnp.dot(a_ref[...], t_ref[...], preferred_element_type=jnp.float32)
    m = jnp.max(s, axis=1, keepdims=True)
    loc = jnp.min(jnp.where(s == m, iota_ref[...], jnp.float32(P)),
                  axis=1, keepdims=True)
    m_ref[...] = m[None]
    l_ref[...] = loc[None]


def _merge_body(m_ref, l_ref, best_ref, idx_ref):
    m = m_ref[...]
    l = l_ref[...]
    best = jnp.max(m, axis=1, keepdims=True)
    idx_ref[...] = jnp.min(jnp.where(m == best, l, jnp.float32(P)),
                           axis=1, keepdims=True)
    best_ref[...] = best


def _sc_gather(table_pad, idxs):
    mesh = plsc.VectorSubcoreMesh(core_axis_name="c", subcore_axis_name="s")

    @functools.partial(
        pl.kernel,
        mesh=mesh,
        out_type=jax.ShapeDtypeStruct((B, 16), jnp.float32),
        scratch_types=[
            pltpu.VMEM((B_PER_W,), jnp.int32),
            pltpu.VMEM((B_PER_W, 16), jnp.float32),
            pltpu.SemaphoreType.DMA,
        ],
        compiler_params=pltpu.CompilerParams(use_tc_tiling_on_sc=False),
    )
    def gather_k(table_hbm, idx_hbm, out_hbm, idx_v, rows_v, sem):
        wid = lax.axis_index("s") * SC_CORES + lax.axis_index("c")
        base = wid * B_PER_W
        pltpu.sync_copy(idx_hbm.at[pl.ds(base, B_PER_W)], idx_v)
        pltpu.async_copy(table_hbm.at[idx_v], rows_v, sem).wait()
        pltpu.sync_copy(rows_v, out_hbm.at[pl.ds(base, B_PER_W)])

    return gather_k(table_pad, idxs)


def kernel(rotMat, output_rotmats):
    a = rotMat.reshape(B, 9)
    t = output_rotmats.reshape(P, 9)
    tt = t.T
    iota_f = jnp.arange(P, dtype=jnp.float32).reshape(1, P)

    m3, l3 = pl.pallas_call(
        _tile_body,
        grid=(NP,),
        in_specs=[
            pl.BlockSpec((B, 9), lambda j: (0, 0)),
            pl.BlockSpec((9, PT), lambda j: (0, j)),
            pl.BlockSpec((1, PT), lambda j: (0, j)),
        ],
        out_specs=[
            pl.BlockSpec((1, B, 1), lambda j: (j, 0, 0)),
            pl.BlockSpec((1, B, 1), lambda j: (j, 0, 0)),
        ],
        out_shape=[
            jax.ShapeDtypeStruct((NP, B, 1), jnp.float32),
            jax.ShapeDtypeStruct((NP, B, 1), jnp.float32),
        ],
    )(a, tt, iota_f)

    m2 = m3.reshape(NP, B).T
    l2 = l3.reshape(NP, B).T

    best, idx = pl.pallas_call(
        _merge_body,
        out_shape=[
            jax.ShapeDtypeStruct((B, 1), jnp.float32),
            jax.ShapeDtypeStruct((B, 1), jnp.float32),
        ],
    )(m2, l2)

    table_pad = jnp.pad(t, ((0, 0), (0, 7)))
    rows = _sc_gather(table_pad, idx.reshape(B).astype(jnp.int32))
    nearest = rows[:, :9].reshape(B, 3, 3)
    return best.reshape(B), nearest


# speedup vs baseline: 1.2544x; 1.2544x over previous
import functools

import jax
import jax.numpy as jnp
from jax import lax
from jax.experimental import pallas as pl
from jax.experimental.pallas import tpu as pltpu
from jax.experimental.pallas import tpu_sc as plsc

B = 4096
P = 36864
PT = 3072
NP = P // PT

SC_CORES = 2
SC_SUBCORES = 16
NW = SC_CORES * SC_SUBCORES
B_PER_W = B // NW


def _argmax_body(a_ref, t_ref, iota_ref, best_ref, idx_ref):
    j = pl.program_id(0)
    s = jnp.dot(a_ref[...], t_ref[...], preferred_element_type=jnp.float32)
    m = jnp.max(s, axis=1, keepdims=True)
    iota = iota_ref[...]
    loc = jnp.min(jnp.where(s == m, iota, jnp.float32(P)), axis=1, keepdims=True)

    @pl.when(j == 0)
    def _():
        best_ref[...] = m
        idx_ref[...] = loc

    @pl.when(j > 0)
    def _():
        prev = best_ref[...]
        upd = m > prev
        best_ref[...] = jnp.where(upd, m, prev)
        idx_ref[...] = jnp.where(upd, loc, idx_ref[...])


def _sc_gather(table_pad, idxs):
    mesh = plsc.VectorSubcoreMesh(core_axis_name="c", subcore_axis_name="s")

    @functools.partial(
        pl.kernel,
        mesh=mesh,
        out_type=jax.ShapeDtypeStruct((B, 16), jnp.float32),
        scratch_types=[
            pltpu.VMEM((B_PER_W,), jnp.int32),
            pltpu.VMEM((B_PER_W, 16), jnp.float32),
            pltpu.SemaphoreType.DMA,
        ],
        compiler_params=pltpu.CompilerParams(use_tc_tiling_on_sc=False),
    )
    def gather_k(table_hbm, idx_hbm, out_hbm, idx_v, rows_v, sem):
        wid = lax.axis_index("s") * SC_CORES + lax.axis_index("c")
        base = wid * B_PER_W
        pltpu.sync_copy(idx_hbm.at[pl.ds(base, B_PER_W)], idx_v)
        pltpu.async_copy(table_hbm.at[idx_v], rows_v, sem).wait()
        pltpu.sync_copy(rows_v, out_hbm.at[pl.ds(base, B_PER_W)])

    return gather_k(table_pad, idxs)


def kernel(rotMat, output_rotmats):
    a = rotMat.reshape(B, 9)
    t = output_rotmats.reshape(P, 9)
    tt = t.T
    iota_f = jnp.arange(P, dtype=jnp.float32).reshape(1, P)

    best, idx = pl.pallas_call(
        _argmax_body,
        grid=(NP,),
        in_specs=[
            pl.BlockSpec((B, 9), lambda j: (0, 0)),
            pl.BlockSpec((9, PT), lambda j: (0, j)),
            pl.BlockSpec((1, PT), lambda j: (0, j)),
        ],
        out_specs=[
            pl.BlockSpec((B, 1), lambda j: (0, 0)),
            pl.BlockSpec((B, 1), lambda j: (0, 0)),
        ],
        out_shape=[
            jax.ShapeDtypeStruct((B, 1), jnp.float32),
            jax.ShapeDtypeStruct((B, 1), jnp.float32),
        ],
    )(a, tt, iota_f)

    table_pad = jnp.pad(t, ((0, 0), (0, 7)))
    rows = _sc_gather(table_pad, idx.reshape(B).astype(jnp.int32))
    nearest = rows[:, :9].reshape(B, 3, 3)
    return best.reshape(B), nearest
